# baseline (speedup 1.0000x reference)
import functools

import jax
import jax.numpy as jnp
from jax import lax
from jax.experimental import pallas as pl
from jax.experimental.pallas import tpu as pltpu
from jax.experimental.pallas import tpu_sc as plsc

NUM_OPERATIONS = 1000000
EMBED_DIM = 64
BATCH = 16384

NC = 2
NS = 16
NW = NC * NS
B_PER_W = BATCH // NW
CHUNK = 128
NCHUNK = B_PER_W // CHUNK
LANES = 16
VPR = EMBED_DIM // LANES


def _rsqrt_newton(x):
    i = lax.bitcast_convert_type(x, jnp.int32)
    i = jnp.int32(0x5F3759DF) - (i >> 1)
    y = lax.bitcast_convert_type(i, jnp.float32)
    for _ in range(3):
        y = y * (1.5 - 0.5 * x * y * y)
    return y


def _lane_sum(v):
    lanes = lax.iota(jnp.int32, LANES)
    for s in (8, 4, 2, 1):
        perm = (lanes + s) % LANES
        v = v + v.at[perm].get(mode="promise_in_bounds")
    return v


def _sc_body(table_hbm, idx_hbm, out_hbm, idx_v, rows_v, sem):
    wid = lax.axis_index("s") * NC + lax.axis_index("c")
    base = wid * B_PER_W

    pltpu.sync_copy(idx_hbm.at[pl.ds(wid * NCHUNK, NCHUNK)], idx_v)

    copies = [
        pltpu.async_copy(
            table_hbm.at[idx_v.at[j]],
            rows_v.at[pl.ds(j * CHUNK, CHUNK)],
            sem,
        )
        for j in range(NCHUNK)
    ]
    for c in copies:
        c.wait()

    def row_body(i, carry):
        vs = [rows_v[i, pl.ds(k * LANES, LANES)] for k in range(VPR)]
        sq = vs[0] * vs[0]
        for k in range(1, VPR):
            sq = sq + vs[k] * vs[k]
        tot = _lane_sum(sq)
        tot = jnp.maximum(tot, jnp.float32(1e-30))
        inv = jnp.minimum(_rsqrt_newton(tot), jnp.float32(1e12))
        for k in range(VPR):
            rows_v[i, pl.ds(k * LANES, LANES)] = vs[k] * inv
        return carry

    lax.fori_loop(0, B_PER_W, row_body, 0, unroll=4)

    pltpu.sync_copy(rows_v, out_hbm.at[pl.ds(base, B_PER_W)])


@functools.lru_cache(maxsize=None)
def _build():
    mesh = plsc.VectorSubcoreMesh(
        core_axis_name="c", subcore_axis_name="s", num_cores=NC, num_subcores=NS
    )
    return pl.kernel(
        _sc_body,
        out_type=jax.ShapeDtypeStruct((BATCH, EMBED_DIM), jnp.float32),
        mesh=mesh,
        scratch_types=[
            pltpu.VMEM((NCHUNK, CHUNK), jnp.int32),
            pltpu.VMEM((B_PER_W, EMBED_DIM), jnp.float32),
            pltpu.SemaphoreType.DMA,
        ],
        compiler_params=pltpu.CompilerParams(use_tc_tiling_on_sc=False),
    )


def kernel(operation_ids, table):
    idx = operation_ids.astype(jnp.int32).reshape(NW * NCHUNK, CHUNK)
    return _build()(table, idx)
